# baseline (speedup 1.0000x reference)
import functools

import jax
import jax.numpy as jnp
from jax import lax
from jax.experimental import pallas as pl
from jax.experimental.pallas import tpu as pltpu
from jax.experimental.pallas import tpu_sc as plsc

E = 320000
D = 128
NC = 2
NS = 16
L = 16
NW = NC * NS
EPW = E // NW
CH = 128
NCHUNK = -(-EPW // CH)
EPAD = NCHUNK * CH
NBUF = 2
NITER = -(-NCHUNK // NBUF)

_SHUF_DNUMS = lax.GatherDimensionNumbers(
    offset_dims=(), collapsed_slice_dims=(0,), start_index_map=(0,))


def _shuffle(x, idx):
    return lax.gather(x, idx[:, None], _SHUF_DNUMS, slice_sizes=(1,),
                      mode=lax.GatherScatterMode.PROMISE_IN_BOUNDS)


def _dot_sigmoid_chunk(rows_s, rows_d, outv, out_base):
    lane = lax.iota(jnp.int32, L)
    perms = [lane ^ d for d in (8, 4, 2, 1)]

    def group(g, _):
        def edge(i, resv):
            e = g * L + i
            acc = rows_s[e, pl.ds(0, L)] * rows_d[e, pl.ds(0, L)]
            for k in range(1, D // L):
                acc = acc + rows_s[e, pl.ds(k * L, L)] * rows_d[e, pl.ds(k * L, L)]
            for p in perms:
                acc = acc + _shuffle(acc, p)
            return jnp.where(lane == i, acc, resv)

        resv = lax.fori_loop(0, L, edge, jnp.zeros((L,), jnp.float32),
                             unroll=8)
        outv[pl.ds(out_base + g * L, L)] = 1.0 / (1.0 + jnp.exp(-resv))
        return 0

    lax.fori_loop(0, CH // L, group, 0)


def _sc_body(z_hbm, zd_hbm, src_hbm, dst_hbm, out_hbm,
             src_ix, dst_ix, srows, drows, outv, *sems):
    wid = lax.axis_index("s") * NC + lax.axis_index("c")
    base = wid * EPW
    pltpu.sync_copy(src_hbm.at[pl.ds(base, EPW)], src_ix.at[pl.ds(0, EPW)])
    pltpu.sync_copy(dst_hbm.at[pl.ds(base, EPW)], dst_ix.at[pl.ds(0, EPW)])
    zero = jnp.zeros((L,), jnp.int32)
    for t in range(EPW, EPAD, L):
        src_ix[pl.ds(t, L)] = zero
        dst_ix[pl.ds(t, L)] = zero

    def start(c, b):
        pltpu.async_copy(z_hbm.at[src_ix.at[pl.ds(c * CH, CH)]],
                         srows.at[b], sems[2 * b])
        pltpu.async_copy(zd_hbm.at[dst_ix.at[pl.ds(c * CH, CH)]],
                         drows.at[b], sems[2 * b + 1])

    def drain(c, b):
        pltpu.make_async_copy(z_hbm.at[src_ix.at[pl.ds(c * CH, CH)]],
                              srows.at[b], sems[2 * b]).wait()
        pltpu.make_async_copy(zd_hbm.at[dst_ix.at[pl.ds(c * CH, CH)]],
                              drows.at[b], sems[2 * b + 1]).wait()

    for b in range(NBUF):
        start(b, b)

    def ring(i, _):
        for b in range(NBUF):
            c = i * NBUF + b

            @pl.when(c < NCHUNK)
            def _():
                drain(c, b)
                _dot_sigmoid_chunk(srows.at[b], drows.at[b], outv, c * CH)

                @pl.when(c + NBUF < NCHUNK)
                def _():
                    start(c + NBUF, b)

        return 0

    lax.fori_loop(0, NITER, ring, 0)
    pltpu.sync_copy(outv.at[pl.ds(0, EPW)], out_hbm.at[pl.ds(base, EPW)])


@jax.jit
def _sc_call(z, zd, src, dst):
    mesh = plsc.VectorSubcoreMesh(core_axis_name="c", subcore_axis_name="s")
    return pl.kernel(
        _sc_body,
        out_type=jax.ShapeDtypeStruct((E,), jnp.float32),
        mesh=mesh,
        scratch_types=[
            pltpu.VMEM((EPAD,), jnp.int32),
            pltpu.VMEM((EPAD,), jnp.int32),
            pltpu.VMEM((NBUF, CH, D), jnp.float32),
            pltpu.VMEM((NBUF, CH, D), jnp.float32),
            pltpu.VMEM((EPAD,), jnp.float32),
        ] + [pltpu.SemaphoreType.DMA] * (2 * NBUF),
    )(z, zd, src, dst)


def kernel(z, zd, edge_idx):
    src = edge_idx[0].astype(jnp.int32)
    dst = edge_idx[1].astype(jnp.int32)
    return _sc_call(z, zd, src, dst)


# speedup vs baseline: 1.8774x; 1.8774x over previous
import functools

import jax
import jax.numpy as jnp
from jax import lax
from jax.experimental import pallas as pl
from jax.experimental.pallas import tpu as pltpu
from jax.experimental.pallas import tpu_sc as plsc

E = 320000
D = 128
NC = 2
NS = 16
L = 16
NW = NC * NS
EPW = E // NW
CH = 80
NCHUNK = EPW // CH
NTRIP = (NCHUNK - 2) // 3


_SHUF_DNUMS = lax.GatherDimensionNumbers(
    offset_dims=(), collapsed_slice_dims=(0,), start_index_map=(0,))


def _shuffle(x, idx):
    return lax.gather(x, idx[:, None], _SHUF_DNUMS, slice_sizes=(1,),
                      mode=lax.GatherScatterMode.PROMISE_IN_BOUNDS)


def _dot_sigmoid_chunk(rows_s, rows_d, outv, out_base):
    lane = lax.iota(jnp.int32, L)
    perms = [lane ^ d for d in (8, 4, 2, 1)]

    def group(g, _):
        def edge(i, resv):
            e = g * L + i
            acc = rows_s[e, pl.ds(0, L)] * rows_d[e, pl.ds(0, L)]
            for k in range(1, D // L):
                acc = acc + rows_s[e, pl.ds(k * L, L)] * rows_d[e, pl.ds(k * L, L)]
            for p in perms:
                acc = acc + _shuffle(acc, p)
            return jnp.where(lane == i, acc, resv)

        resv = lax.fori_loop(0, L, edge, jnp.zeros((L,), jnp.float32),
                             unroll=8)
        outv[pl.ds(out_base + g * L, L)] = 1.0 / (1.0 + jnp.exp(-resv))
        return 0

    lax.fori_loop(0, CH // L, group, 0)


def _sc_body(z_hbm, zd_hbm, src_hbm, dst_hbm, out_hbm,
             src_ix, dst_ix, srows, drows, outv, *semlist):
    wid = lax.axis_index("s") * NC + lax.axis_index("c")
    base = wid * EPW
    pltpu.sync_copy(src_hbm.at[pl.ds(base, EPW)], src_ix)
    pltpu.sync_copy(dst_hbm.at[pl.ds(base, EPW)], dst_ix)

    sems = tuple((semlist[2 * b], semlist[2 * b + 1]) for b in range(3))

    def start(c, b):
        pltpu.async_copy(z_hbm.at[src_ix.at[pl.ds(c * CH, CH)]],
                         srows.at[b], sems[b][0])
        pltpu.async_copy(zd_hbm.at[dst_ix.at[pl.ds(c * CH, CH)]],
                         drows.at[b], sems[b][1])

    def drain(c, b):
        pltpu.make_async_copy(z_hbm.at[src_ix.at[pl.ds(c * CH, CH)]],
                              srows.at[b], sems[b][0]).wait()
        pltpu.make_async_copy(zd_hbm.at[dst_ix.at[pl.ds(c * CH, CH)]],
                              drows.at[b], sems[b][1]).wait()

    start(0, 0)
    start(1, 1)
    start(2, 2)

    def trip(i, _):
        c = 3 * i
        for b in range(3):
            drain(c + b, b)
            _dot_sigmoid_chunk(srows.at[b], drows.at[b], outv, (c + b) * CH)
            if b < 2:
                start(c + b + 3, b)
            else:
                @pl.when(c + b + 3 < NCHUNK)
                def _():
                    start(c + b + 3, b)

        return 0

    lax.fori_loop(0, NTRIP, trip, 0)

    for b in range(2):
        drain(NCHUNK - 2 + b, b)
        _dot_sigmoid_chunk(srows.at[b], drows.at[b], outv,
                           (NCHUNK - 2 + b) * CH)

    pltpu.sync_copy(outv, out_hbm.at[pl.ds(base, EPW)])


@jax.jit
def _sc_call(z, zd, src, dst):
    mesh = plsc.VectorSubcoreMesh(core_axis_name="c", subcore_axis_name="s")
    return pl.kernel(
        _sc_body,
        out_type=jax.ShapeDtypeStruct((E,), jnp.float32),
        mesh=mesh,
        scratch_types=[
            pltpu.VMEM((EPW,), jnp.int32),
            pltpu.VMEM((EPW,), jnp.int32),
            pltpu.VMEM((3, CH, D), jnp.float32),
            pltpu.VMEM((3, CH, D), jnp.float32),
            pltpu.VMEM((EPW,), jnp.float32),
        ] + [pltpu.SemaphoreType.DMA] * 6,
    )(z, zd, src, dst)


def kernel(z, zd, edge_idx):
    src = edge_idx[0].astype(jnp.int32)
    dst = edge_idx[1].astype(jnp.int32)
    return _sc_call(z, zd, src, dst)
